# baseline (speedup 1.0000x reference)
import functools

import jax
import jax.numpy as jnp
from jax import lax
from jax.experimental import pallas as pl
from jax.experimental.pallas import tpu as pltpu
from jax.experimental.pallas import tpu_sc as plsc

N_IN = 100000
N_OUT = 100000
C = 128
KVOL = 27
P = 16384
TOTAL = KVOL * P
NW = 32
L = 16

ROWS_PER_W = TOTAL // NW
GB = 128
N_GB = ROWS_PER_W // GB

CH = 8192
NCHUNK = 13
CPS = 7
ACC_ROWS = CH + 128
PPT = TOTAL // 16
NVEC = PPT // L
FB = 128
SEC = PPT // 4
RPT = 16 * NCHUNK * NVEC + 128
LAST_FULL = (N_OUT - (NCHUNK - 1) * CH) // FB
LAST_TAIL = N_OUT - (NCHUNK - 1) * CH - LAST_FULL * FB

_mesh = lambda: plsc.VectorSubcoreMesh(core_axis_name="c", subcore_axis_name="s")


@functools.partial(
    pl.kernel,
    out_type=jax.ShapeDtypeStruct((TOTAL, C), jnp.float32),
    mesh=_mesh(),
    scratch_types=[
        pltpu.VMEM((GB,), jnp.int32),
        pltpu.VMEM((GB, C), jnp.float32),
        pltpu.SemaphoreType.DMA,
    ],
)
def _sc_gather(feat_hbm, imap_hbm, gath_hbm, idx_v, rows_v, sem):
    cid = lax.axis_index("c")
    sid = lax.axis_index("s")
    wid = sid * 2 + cid
    base = wid * ROWS_PER_W

    def body(b, carry):
        off = base + b * GB
        pltpu.sync_copy(imap_hbm.at[pl.ds(off, GB)], idx_v)
        pltpu.async_copy(feat_hbm.at[idx_v], rows_v, sem).wait()
        pltpu.sync_copy(rows_v, gath_hbm.at[pl.ds(off, GB)])
        return carry

    lax.fori_loop(0, N_GB, body, jnp.int32(0))


def _gemm_body(x_ref, w_ref, o_ref):
    o_ref[0] = jnp.dot(x_ref[0], w_ref[0], preferred_element_type=jnp.float32)


def _tc_gemm(gathered, weights):
    TM = 1024
    return pl.pallas_call(
        _gemm_body,
        grid=(KVOL, P // TM),
        in_specs=[
            pl.BlockSpec((1, TM, C), lambda k, i: (k, i, 0)),
            pl.BlockSpec((1, C, C), lambda k, i: (k, 0, 0)),
        ],
        out_specs=pl.BlockSpec((1, TM, C), lambda k, i: (k, i, 0)),
        out_shape=jax.ShapeDtypeStruct((KVOL, P, C), jnp.float32),
        compiler_params=pltpu.CompilerParams(
            dimension_semantics=("arbitrary", "arbitrary")),
    )(gathered, weights)


@functools.partial(
    pl.kernel,
    out_type=(jax.ShapeDtypeStruct((N_OUT, C), jnp.float32),
              jax.ShapeDtypeStruct((NW * RPT,), jnp.int32)),
    mesh=_mesh(),
    scratch_types=[
        pltpu.VMEM((PPT + FB,), jnp.int32),
        pltpu.VMEM((SEC,), jnp.int32),
        pltpu.VMEM((SEC,), jnp.int32),
        pltpu.VMEM((FB,), jnp.int32),
        pltpu.VMEM((FB,), jnp.int32),
        pltpu.VMEM((FB,), jnp.int32),
        pltpu.VMEM((FB, C), jnp.float32),
        pltpu.VMEM_SHARED((ACC_ROWS, C), jnp.float32),
        pltpu.SemaphoreType.DMA,
    ],
)
def _sc_scatter(omap_hbm, contrib_hbm, out_hbm, reg_hbm,
                omap_v, payl, dest, dstage, pstage, lstage, rows, acc, sem):
    cid = lax.axis_index("c")
    sid = lax.axis_index("s")
    ps = sid * PPT
    wid = sid * 2 + cid
    wbase = wid * RPT
    iota = lax.iota(jnp.int32, L)
    psv = jnp.zeros((L,), jnp.int32) + ps
    c_lo = cid * CPS
    c_hi = jnp.minimum(c_lo + CPS, NCHUNK)

    def dfill(v, carry):
        slot = v * L + iota
        dval = jnp.bitwise_or(lax.shift_left(jnp.bitwise_and(slot, 8191), 14),
                              CH + jnp.bitwise_and(slot, 127))
        payl[pl.ds(v * L, L)] = dval
        return carry

    lax.fori_loop(0, SEC // L, dfill, jnp.int32(0))

    def pf_body(r, carry):
        pltpu.sync_copy(payl.at[pl.ds(0, SEC)],
                        reg_hbm.at[pl.ds(wbase + r * SEC, SEC)])
        return carry

    lax.fori_loop(0, (16 * NCHUNK * NVEC) // SEC, pf_body, jnp.int32(0))

    pltpu.sync_copy(omap_hbm.at[pl.ds(ps, PPT)], omap_v.at[pl.ds(0, PPT)])

    dumpb = wbase + 16 * NCHUNK * NVEC

    def make_bin(sec):
        def bin_body(v, carry):
            relv = carry[0]
            cnts = carry[1:]
            idx = omap_v[pl.ds(sec * SEC + v * L, L)]
            c = jnp.right_shift(idx, 13)
            pk = jnp.bitwise_or(lax.shift_left(relv, 14),
                                jnp.bitwise_and(idx, 8191))
            owned = jnp.logical_and(c >= c_lo, c < c_hi)
            slot = jnp.zeros((L,), jnp.int32)
            new_cnts = []
            for cc in range(NCHUNK):
                hit = c == cc
                slot = jnp.where(hit, cnts[cc], slot)
                new_cnts.append(cnts[cc] + jnp.where(hit, 1, 0))
            d = wbase + (c * 16 + iota) * NVEC + slot
            d = jnp.where(owned, d, dumpb + jnp.bitwise_and(relv, 127))
            payl[pl.ds(v * L, L)] = pk
            dest[pl.ds(v * L, L)] = d
            return tuple([relv + L] + new_cnts)
        return bin_body

    def sc_body(b, carry):
        for j in range(FB // L):
            dstage[pl.ds(j * L, L)] = dest[pl.ds(b * FB + j * L, L)]
        pltpu.sync_copy(payl.at[pl.ds(b * FB, FB)], reg_hbm.at[dstage])
        return carry

    carry = tuple([iota] + [jnp.zeros((L,), jnp.int32)] * NCHUNK)
    for sec in range(PPT // SEC):
        carry = lax.fori_loop(0, SEC // L, make_bin(sec), carry)
        lax.fori_loop(0, SEC // FB, sc_body, jnp.int32(0))
    cnts = carry[1:]

    zv = jnp.zeros((L,), jnp.float32)
    for ch in range(CPS):
        chunk = c_lo + ch
        valid = chunk < NCHUNK
        lo = chunk * CH

        def zb(t, carry):
            rows[t // (C // L), pl.ds((t % (C // L)) * L, L)] = zv
            return carry

        lax.fori_loop(0, FB * (C // L), zb, jnp.int32(0))
        for b in range(5):
            blk = sid + b * 16

            @pl.when(jnp.logical_and(valid, blk < ACC_ROWS // FB))
            def _zero():
                pltpu.sync_copy(rows, acc.at[pl.ds(blk * FB, FB)])

        plsc.subcore_barrier()

        cntv = jnp.zeros((L,), jnp.int32)
        for cc in range(NCHUNK):
            cntv = jnp.where(chunk == cc, cnts[cc], cntv)

        total = jnp.int32(0)
        for l in range(16):
            rcnt = jnp.bitwise_and(cntv[l] + 7, ~jnp.int32(7))
            rbase = wbase + (chunk * 16 + l) * NVEC
            t0 = total

            def cp(i, carry):
                pltpu.sync_copy(reg_hbm.at[pl.ds(pl.multiple_of(rbase + i * FB, 8), FB)],
                                omap_v.at[pl.ds(pl.multiple_of(t0 + i * FB, 8), FB)])
                return carry

            n128 = lax.div(rcnt, jnp.int32(FB))
            lax.fori_loop(0, n128, cp, jnp.int32(0))
            rem = jnp.bitwise_and(rcnt, FB - 1)
            o2 = t0 + n128 * FB
            r2 = rbase + n128 * FB
            a64 = jnp.bitwise_and(rem, 64)
            a32 = jnp.bitwise_and(rem, 32)
            a16 = jnp.bitwise_and(rem, 16)

            @pl.when(a64 != 0)
            def _c64():
                pltpu.sync_copy(reg_hbm.at[pl.ds(pl.multiple_of(r2, 8), 64)],
                                omap_v.at[pl.ds(pl.multiple_of(o2, 8), 64)])

            @pl.when(a32 != 0)
            def _c32():
                pltpu.sync_copy(reg_hbm.at[pl.ds(pl.multiple_of(r2 + a64, 8), 32)],
                                omap_v.at[pl.ds(pl.multiple_of(o2 + a64, 8), 32)])

            @pl.when(a16 != 0)
            def _c16():
                pltpu.sync_copy(reg_hbm.at[pl.ds(pl.multiple_of(r2 + a64 + a32, 8), 16)],
                                omap_v.at[pl.ds(pl.multiple_of(o2 + a64 + a32, 8), 16)])

            @pl.when(jnp.bitwise_and(rcnt, 8) != 0)
            def _c8():
                pltpu.sync_copy(
                    reg_hbm.at[pl.ds(pl.multiple_of(r2 + a64 + a32 + a16, 8), 8)],
                    omap_v.at[pl.ds(pl.multiple_of(o2 + a64 + a32 + a16, 8), 8)])

            total = total + rcnt

        for j in range(FB // L):
            slot = iota + j * L
            omap_v[pl.ds(pl.multiple_of(total + j * L, 8), L)] = jnp.bitwise_or(
                lax.shift_left(jnp.bitwise_and(slot, 8191), 14),
                CH + jnp.bitwise_and(slot, 127))

        nb = lax.div(total + (FB - 1), jnp.int32(FB))

        def bbody(b, carry):
            for j in range(FB // L):
                pk = omap_v[pl.ds(b * FB + j * L, L)]
                pstage[pl.ds(j * L, L)] = psv + jnp.right_shift(pk, 14)
                lstage[pl.ds(j * L, L)] = jnp.bitwise_and(pk, 16383)
            pltpu.async_copy(contrib_hbm.at[pstage], rows, sem).wait()
            pltpu.sync_copy(rows, acc.at[lstage], add=True)
            return carry

        lax.fori_loop(0, jnp.where(valid, nb, 0), bbody, jnp.int32(0))
        plsc.subcore_barrier()

        for b in range(4):
            blk = sid + b * 16
            full = jnp.logical_and(
                valid,
                jnp.logical_or(chunk < NCHUNK - 1, blk < LAST_FULL))

            @pl.when(full)
            def _wout():
                pltpu.sync_copy(acc.at[pl.ds(blk * FB, FB)],
                                out_hbm.at[pl.ds(lo + blk * FB, FB)])

        @pl.when(jnp.logical_and(sid == 15, chunk == NCHUNK - 1))
        def _tail():
            pltpu.sync_copy(
                acc.at[pl.ds(LAST_FULL * FB, LAST_TAIL)],
                out_hbm.at[pl.ds(lo + LAST_FULL * FB, LAST_TAIL)])

        plsc.subcore_barrier()


def kernel(features, kernel, in_map, out_map):
    imap = in_map.reshape(-1).astype(jnp.int32)
    omap = out_map.reshape(-1).astype(jnp.int32)
    gathered = _sc_gather(features, imap)
    contrib = _tc_gemm(gathered.reshape(KVOL, P, C), kernel)
    out, _ = _sc_scatter(omap, contrib.reshape(TOTAL, C))
    return out


# speedup vs baseline: 1.0141x; 1.0141x over previous
import functools

import jax
import jax.numpy as jnp
from jax import lax
from jax.experimental import pallas as pl
from jax.experimental.pallas import tpu as pltpu
from jax.experimental.pallas import tpu_sc as plsc

N_IN = 100000
N_OUT = 100000
C = 128
KVOL = 27
P = 16384
TOTAL = KVOL * P
NW = 32
L = 16

ROWS_PER_W = TOTAL // NW
GB = 128
N_GB = ROWS_PER_W // GB

CH = 8192
NCHUNK = 13
CPS = 7
ACC_ROWS = CH + 128
PPT = TOTAL // 16
NVEC = PPT // L
FB = 128
SEC = PPT // 24
RPT = 16 * NCHUNK * NVEC + 128
LAST_FULL = (N_OUT - (NCHUNK - 1) * CH) // FB
LAST_TAIL = N_OUT - (NCHUNK - 1) * CH - LAST_FULL * FB

_mesh = lambda: plsc.VectorSubcoreMesh(core_axis_name="c", subcore_axis_name="s")


@functools.partial(
    pl.kernel,
    out_type=jax.ShapeDtypeStruct((TOTAL, C), jnp.float32),
    mesh=_mesh(),
    scratch_types=[
        pltpu.VMEM((N_GB, GB), jnp.int32),
        pltpu.VMEM((GB, C), jnp.float32),
        pltpu.VMEM((GB, C), jnp.float32),
        pltpu.SemaphoreType.DMA,
        pltpu.SemaphoreType.DMA,
        pltpu.SemaphoreType.DMA,
        pltpu.SemaphoreType.DMA,
    ],
)
def _sc_gather(feat_hbm, imap_hbm, gath_hbm, idx_v, rows_a, rows_b,
               sa, sb, swa, swb):
    cid = lax.axis_index("c")
    sid = lax.axis_index("s")
    wid = sid * 2 + cid
    base = wid * ROWS_PER_W

    pltpu.sync_copy(imap_hbm.at[wid], idx_v)

    def body(i, carry):
        b0 = 2 * i
        da = pltpu.async_copy(feat_hbm.at[idx_v.at[b0]], rows_a, sa)
        db = pltpu.async_copy(feat_hbm.at[idx_v.at[b0 + 1]], rows_b, sb)
        da.wait()
        wa = pltpu.async_copy(
            rows_a, gath_hbm.at[pl.ds(base + b0 * GB, GB)], swa)
        db.wait()
        wb = pltpu.async_copy(
            rows_b, gath_hbm.at[pl.ds(base + (b0 + 1) * GB, GB)], swb)
        wa.wait()
        wb.wait()
        return carry

    lax.fori_loop(0, N_GB // 2, body, jnp.int32(0))


def _gemm_body(x_ref, w_ref, o_ref):
    o_ref[0] = jnp.dot(x_ref[0], w_ref[0], preferred_element_type=jnp.float32)


def _tc_gemm(gathered, weights):
    TM = 1024
    return pl.pallas_call(
        _gemm_body,
        grid=(KVOL, P // TM),
        in_specs=[
            pl.BlockSpec((1, TM, C), lambda k, i: (k, i, 0)),
            pl.BlockSpec((1, C, C), lambda k, i: (k, 0, 0)),
        ],
        out_specs=pl.BlockSpec((1, TM, C), lambda k, i: (k, i, 0)),
        out_shape=jax.ShapeDtypeStruct((KVOL, P, C), jnp.float32),
        compiler_params=pltpu.CompilerParams(
            dimension_semantics=("arbitrary", "arbitrary")),
    )(gathered, weights)


@functools.partial(
    pl.kernel,
    out_type=(jax.ShapeDtypeStruct((N_OUT, C), jnp.float32),
              jax.ShapeDtypeStruct((NW * RPT,), jnp.int32)),
    mesh=_mesh(),
    scratch_types=[
        pltpu.VMEM((PPT + 2 * FB,), jnp.int32),
        pltpu.VMEM((SEC,), jnp.int32),
        pltpu.VMEM((SEC,), jnp.int32),
        pltpu.VMEM((FB,), jnp.int32),
        pltpu.VMEM((FB,), jnp.int32),
        pltpu.VMEM((FB,), jnp.int32),
        pltpu.VMEM((FB,), jnp.int32),
        pltpu.VMEM((FB, C), jnp.float32),
        pltpu.VMEM((FB, C), jnp.float32),
        pltpu.VMEM_SHARED((ACC_ROWS, C), jnp.float32),
        pltpu.SemaphoreType.DMA,
        pltpu.SemaphoreType.DMA,
    ],
)
def _sc_scatter(omap_hbm, contrib_hbm, out_hbm, reg_hbm,
                omap_v, payl, dest, dstage, pstage, pstage_b, lstage,
                rows, rows_b, acc, sem, sem_b):
    cid = lax.axis_index("c")
    sid = lax.axis_index("s")
    ps = sid * PPT
    wid = sid * 2 + cid
    wbase = wid * RPT
    iota = lax.iota(jnp.int32, L)
    psv = jnp.zeros((L,), jnp.int32) + ps
    c_lo = cid * CPS
    c_hi = jnp.minimum(c_lo + CPS, NCHUNK)

    def dfill(v, carry):
        slot = v * L + iota
        payl[pl.ds(v * L, L)] = jnp.bitwise_or(
            lax.shift_left(jnp.bitwise_and(slot, 8191), 14),
            CH + jnp.bitwise_and(slot, 127))
        return carry

    lax.fori_loop(0, SEC // L, dfill, jnp.int32(0))

    def pfill(r, carry):
        pltpu.sync_copy(payl, reg_hbm.at[pl.ds(wbase + r * SEC, SEC)])
        return carry

    lax.fori_loop(0, (16 * NCHUNK * NVEC) // SEC, pfill, jnp.int32(0))

    pltpu.sync_copy(omap_hbm.at[pl.ds(ps, PPT)], omap_v.at[pl.ds(0, PPT)])

    dumpb = wbase + 16 * NCHUNK * NVEC

    def make_bin(sec):
        def bin_body(v, carry):
            relv = carry[0]
            cnts = carry[1:]
            idx = omap_v[pl.ds(sec * SEC + v * L, L)]
            c = jnp.right_shift(idx, 13)
            pk = jnp.bitwise_or(lax.shift_left(relv, 14),
                                jnp.bitwise_and(idx, 8191))
            owned = jnp.logical_and(c >= c_lo, c < c_hi)
            slot = jnp.zeros((L,), jnp.int32)
            new_cnts = []
            for cc in range(NCHUNK):
                hit = c == cc
                slot = jnp.where(hit, cnts[cc], slot)
                new_cnts.append(cnts[cc] + jnp.where(hit, 1, 0))
            d = wbase + (c * 16 + iota) * NVEC + slot
            d = jnp.where(owned, d, dumpb + jnp.bitwise_and(relv, 127))
            payl[pl.ds(v * L, L)] = pk
            dest[pl.ds(v * L, L)] = d
            return tuple([relv + L] + new_cnts)
        return bin_body

    def sc_body(b, carry):
        for j in range(FB // L):
            dstage[pl.ds(j * L, L)] = dest[pl.ds(b * FB + j * L, L)]
        pltpu.sync_copy(payl.at[pl.ds(b * FB, FB)], reg_hbm.at[dstage])
        return carry

    def section_body(sec, carry0):
        carry0 = lax.fori_loop(0, SEC // L, make_bin(sec), carry0)
        lax.fori_loop(0, SEC // FB, sc_body, jnp.int32(0))
        return carry0

    carry = lax.fori_loop(
        0, PPT // SEC, section_body,
        tuple([iota] + [jnp.zeros((L,), jnp.int32)] * NCHUNK))
    cnts = carry[1:]

    zv = jnp.zeros((L,), jnp.float32)
    for ch in range(CPS):
        chunk = c_lo + ch
        valid = chunk < NCHUNK
        lo = chunk * CH

        def zb(t, carry):
            rows[t // (C // L), pl.ds((t % (C // L)) * L, L)] = zv
            return carry

        lax.fori_loop(0, FB * (C // L), zb, jnp.int32(0))
        for b in range(5):
            blk = sid + b * 16

            @pl.when(jnp.logical_and(valid, blk < ACC_ROWS // FB))
            def _zero():
                pltpu.sync_copy(rows, acc.at[pl.ds(blk * FB, FB)])

        plsc.subcore_barrier()

        cntv = jnp.zeros((L,), jnp.int32)
        for cc in range(NCHUNK):
            cntv = jnp.where(chunk == cc, cnts[cc], cntv)

        total = jnp.int32(0)
        for l in range(16):
            rcnt = jnp.bitwise_and(cntv[l] + 63, ~jnp.int32(63))
            rbase = wbase + (chunk * 16 + l) * NVEC
            t0 = total

            def cp(i, carry):
                pltpu.sync_copy(reg_hbm.at[pl.ds(pl.multiple_of(rbase + i * FB, 8), FB)],
                                omap_v.at[pl.ds(pl.multiple_of(t0 + i * FB, 8), FB)])
                return carry

            n128 = lax.div(rcnt, jnp.int32(FB))
            lax.fori_loop(0, n128, cp, jnp.int32(0))
            rem = jnp.bitwise_and(rcnt, FB - 1)
            o2 = t0 + n128 * FB
            r2 = rbase + n128 * FB
            a64 = jnp.bitwise_and(rcnt, 64)

            @pl.when(a64 != 0)
            def _c64():
                pltpu.sync_copy(reg_hbm.at[pl.ds(pl.multiple_of(r2, 8), 64)],
                                omap_v.at[pl.ds(pl.multiple_of(o2, 8), 64)])

            total = total + rcnt

        def padb(j, carry):
            slot = iota + j * L
            omap_v[pl.ds(pl.multiple_of(total + j * L, 8), L)] = jnp.bitwise_or(
                lax.shift_left(jnp.bitwise_and(slot, 8191), 14),
                CH + jnp.bitwise_and(slot, 127))
            return carry

        lax.fori_loop(0, FB // L, padb, jnp.int32(0))

        nb = jnp.where(valid, lax.div(total + (FB - 1), jnp.int32(FB)), 0)
        npair = lax.div(nb + 1, jnp.int32(2))

        def bbody(i, carry):
            b0 = 2 * i
            b1 = b0 + 1
            has_b1 = b1 < nb
            for j in range(FB // L):
                pk = omap_v[pl.ds(b0 * FB + j * L, L)]
                pos = psv + jnp.right_shift(pk, 14)
                pos = jnp.minimum(jnp.maximum(pos, 0), TOTAL - 1)
                pstage[pl.ds(j * L, L)] = pos
            da = pltpu.async_copy(contrib_hbm.at[pstage], rows, sem)
            for j in range(FB // L):
                pk = omap_v[pl.ds(b1 * FB + j * L, L)]
                pos = psv + jnp.right_shift(pk, 14)
                pos = jnp.minimum(jnp.maximum(pos, 0), TOTAL - 1)
                pstage_b[pl.ds(j * L, L)] = pos
            db = pltpu.async_copy(contrib_hbm.at[pstage_b], rows_b, sem_b)
            da.wait()
            for j in range(FB // L):
                pk = omap_v[pl.ds(b0 * FB + j * L, L)]
                loc = jnp.minimum(jnp.bitwise_and(pk, 16383), ACC_ROWS - 1)
                lstage[pl.ds(j * L, L)] = loc
            pltpu.sync_copy(rows, acc.at[lstage], add=True)
            db.wait()

            @pl.when(has_b1)
            def _doneb():
                for j in range(FB // L):
                    pk = omap_v[pl.ds(b1 * FB + j * L, L)]
                    loc = jnp.minimum(jnp.bitwise_and(pk, 16383), ACC_ROWS - 1)
                    lstage[pl.ds(j * L, L)] = loc
                pltpu.sync_copy(rows_b, acc.at[lstage], add=True)
            return carry

        lax.fori_loop(0, npair, bbody, jnp.int32(0))
        plsc.subcore_barrier()

        for b in range(4):
            blk = sid + b * 16
            full = jnp.logical_and(
                valid,
                jnp.logical_or(chunk < NCHUNK - 1, blk < LAST_FULL))

            @pl.when(full)
            def _wout():
                pltpu.sync_copy(acc.at[pl.ds(blk * FB, FB)],
                                out_hbm.at[pl.ds(lo + blk * FB, FB)])

        @pl.when(jnp.logical_and(sid == 15, chunk == NCHUNK - 1))
        def _tail():
            pltpu.sync_copy(
                acc.at[pl.ds(LAST_FULL * FB, LAST_TAIL)],
                out_hbm.at[pl.ds(lo + LAST_FULL * FB, LAST_TAIL)])

        plsc.subcore_barrier()


def kernel(features, kernel, in_map, out_map):
    imap = in_map.reshape(NW, N_GB, GB).astype(jnp.int32)
    omap = out_map.reshape(-1).astype(jnp.int32)
    gathered = _sc_gather(features, imap)
    contrib = _tc_gemm(gathered.reshape(KVOL, P, C), kernel)
    out, _ = _sc_scatter(omap, contrib.reshape(TOTAL, C))
    return out
